# baseline (speedup 1.0000x reference)
import functools

import jax
import jax.numpy as jnp
from jax import lax
from jax.experimental import pallas as pl
from jax.experimental.pallas import tpu as pltpu
from jax.experimental.pallas import tpu_sc as plsc

VOCAB = 1_000_000
WORLD = 4
MY_RANK = 1
PART = VOCAB // WORLD
LO = MY_RANK * PART
HI = LO + PART
EMB = 64
BATCH = 4096
SEQ = 200
NTOK = BATCH * SEQ

NC = 2
NS = 16
NW = NC * NS
NB = BATCH // NW
PER_W = NB * SEQ
L = 16
G = PER_W // L
DUMMY = NB


@functools.partial(
    pl.kernel,
    out_type=jax.ShapeDtypeStruct((SEQ, EMB // 8, BATCH // 128, 8, 128),
                                  jnp.float32),
    mesh=plsc.VectorSubcoreMesh(core_axis_name="c", subcore_axis_name="s"),
    compiler_params=pltpu.CompilerParams(
        use_tc_tiling_on_sc=False, needs_layout_passes=False),
    scratch_types=[
        pltpu.VMEM((PER_W,), jnp.int32),
        pltpu.VMEM((PER_W,), jnp.int32),
        pltpu.VMEM((PER_W,), jnp.int32),
        pltpu.VMEM((NB + 1, EMB), jnp.float32),
        pltpu.VMEM((NB + 1, EMB), jnp.float32),
        pltpu.VMEM((EMB // 8, 8, 128), jnp.float32),
        pltpu.VMEM((EMB // 8, 8, 128), jnp.float32),
        pltpu.SemaphoreType.DMA,
        pltpu.SemaphoreType.DMA,
        pltpu.SemaphoreType.DMA,
        pltpu.SemaphoreType.DMA,
    ],
)
def _sc_gather(ids_hbm, tab_hbm, out_hbm, idv, sidT, bposT,
               r0, r1, x0, x1, gs0, gs1, ws0, ws1):
    wid = lax.axis_index("s") * NC + lax.axis_index("c")
    base = wid * PER_W

    pltpu.sync_copy(ids_hbm.at[pl.ds(base, PER_W)], idv)

    zv = jnp.zeros((L,), jnp.float32)
    for k in range(EMB // L):
        r0[DUMMY, pl.ds(k * L, L)] = zv
        r1[DUMMY, pl.ds(k * L, L)] = zv

    ii = lax.iota(jnp.int32, L)

    def prep(g, c2):
        t = g * L + ii
        v = idv[pl.ds(g * L, L)]
        m = (v >= LO) & (v < HI)
        sid = jnp.where(m, v - LO, (v >> 2) & 131071)
        b = (t * 5243) >> 20
        s = t - b * 200
        dest = s * 128 + b
        bpos = jnp.where(m, b, DUMMY)
        plsc.store_scatter(sidT, [dest], sid)
        plsc.store_scatter(bposT, [dest], bpos)
        return c2

    lax.fori_loop(0, G, prep, 0)

    def gat(s, rb, sb):
        pltpu.async_copy(tab_hbm.at[sidT.at[pl.ds(s * NB, NB)]],
                         rb.at[pl.ds(0, NB)], sb)

    def wat_g(rb, sb):
        pltpu.make_async_copy(tab_hbm.at[sidT.at[pl.ds(0, NB)]],
                              rb.at[pl.ds(0, NB)], sb).wait()

    def wrt(s, xb, sb):
        pltpu.async_copy(xb, out_hbm.at[s, :, wid], sb)

    def wat_w(xb, sb):
        pltpu.make_async_copy(xb, out_hbm.at[0, :, wid], sb).wait()

    def transpose(s, rb, xb):
        def bg_body(bg, c2):
            bidx = bposT[pl.ds(s * 128 + bg * L, L)]
            for e in range(EMB):
                col = jnp.full((L,), e, jnp.int32)
                xb[e >> 3, e & 7, pl.ds(bg * L, L)] = plsc.load_gather(
                    rb, [bidx, col])
            return c2

        lax.fori_loop(0, NB // L, bg_body, 0)

    gat(0, r0, gs0)
    gat(1, r1, gs1)

    def step(g, c2):
        s0 = 2 * g
        s1 = 2 * g + 1

        wat_g(r0, gs0)

        @pl.when(s0 >= 2)
        def _():
            wat_w(x0, ws0)

        transpose(s0, r0, x0)
        wrt(s0, x0, ws0)

        @pl.when(s0 + 2 < SEQ)
        def _():
            gat(s0 + 2, r0, gs0)

        wat_g(r1, gs1)

        @pl.when(s1 >= 2)
        def _():
            wat_w(x1, ws1)

        transpose(s1, r1, x1)
        wrt(s1, x1, ws1)

        @pl.when(s1 + 2 < SEQ)
        def _():
            gat(s1 + 2, r1, gs1)

        return c2

    lax.fori_loop(0, SEQ // 2, step, 0)

    wat_w(x0, ws0)
    wat_w(x1, ws1)


def kernel(input_ids, tr):
    ids = input_ids.reshape(NTOK)
    x = _sc_gather(ids, tr)
    return x.transpose(2, 4, 0, 1, 3).reshape(BATCH, SEQ, EMB)
